# baseline (speedup 1.0000x reference)
import functools

import jax
import jax.numpy as jnp
from jax import lax
from jax.experimental import pallas as pl
from jax.experimental.pallas import tpu as pltpu
from jax.experimental.pallas import tpu_sc as plsc

N_USERS = 1000000
N_ASPECTS = 8
BATCH = 16384
A_MIN, A_MAX = 1.0, 5.0
R_MIN, R_MAX = 1.0, 5.0

_NC = 2
_NS = 16
_NW = _NC * _NS
_BPW = BATCH // _NW
_ELEMS = _BPW * N_ASPECTS
_CHUNK = 128
_NCHUNK = _ELEMS // _CHUNK
_CPA = _BPW // _CHUNK
_GROUPS = _BPW // 16


@functools.partial(
    pl.kernel,
    mesh=plsc.VectorSubcoreMesh(core_axis_name="c", subcore_axis_name="s"),
    out_type=jax.ShapeDtypeStruct((BATCH,), jnp.float32),
    compiler_params=pltpu.CompilerParams(
        needs_layout_passes=False, use_tc_tiling_on_sc=False
    ),
    scratch_types=[
        pltpu.VMEM((_BPW,), jnp.int32),
        pltpu.VMEM((_ELEMS,), jnp.int32),
        pltpu.VMEM((_ELEMS,), jnp.float32),
        pltpu.VMEM((_ELEMS,), jnp.float32),
        pltpu.VMEM((_BPW,), jnp.float32),
        pltpu.SemaphoreType.DMA,
    ],
)
def _sc_predict(u_hbm, a_hbm, table_hbm, out_hbm, u_v, idx_v, w_v, a_v, out_v, sem):
    wid = lax.axis_index("s") * _NC + lax.axis_index("c")
    base = wid * _BPW

    pltpu.sync_copy(u_hbm.at[pl.ds(base, _BPW)], u_v)

    copies = []
    for c in range(_NCHUNK):
        j = c // _CPA
        ibase = (c % _CPA) * _CHUNK
        for t in range(_CHUNK // 16):
            u16 = u_v[pl.ds(ibase + t * 16, 16)]
            idx_v[pl.ds(c * _CHUNK + t * 16, 16)] = u16 * N_ASPECTS + j
        copies.append(
            pltpu.async_copy(
                table_hbm.at[idx_v.at[pl.ds(c * _CHUNK, _CHUNK)]],
                w_v.at[pl.ds(c * _CHUNK, _CHUNK)],
                sem,
            )
        )
    for j in range(N_ASPECTS):
        pltpu.sync_copy(
            a_hbm.at[pl.ds(j * BATCH + base, _BPW)], a_v.at[pl.ds(j * _BPW, _BPW)]
        )
    for cp in copies:
        cp.wait()

    def group_body(g, _):
        o = g * 16
        acc = jnp.zeros((16,), jnp.float32)
        l1 = jnp.zeros((16,), jnp.float32)
        for j in range(N_ASPECTS):
            w = w_v[pl.ds(j * _BPW + o, 16)]
            a = a_v[pl.ds(j * _BPW + o, 16)]
            acc = acc + w * a
            l1 = l1 + jnp.abs(w)
        pred = acc / jnp.maximum(l1, 1e-12)
        out_v[pl.ds(o, 16)] = R_MIN + (R_MIN - R_MAX) * (
            (pred - A_MIN) / (A_MAX - A_MIN)
        )
        return 0

    lax.fori_loop(0, _GROUPS, group_body, 0)
    pltpu.sync_copy(out_v, out_hbm.at[pl.ds(base, _BPW)])


def kernel(U_ids, A_ratings, users_parameters):
    return _sc_predict(
        U_ids.astype(jnp.int32),
        A_ratings.reshape(-1),
        users_parameters.reshape(-1),
    )


# speedup vs baseline: 9.4507x; 9.4507x over previous
import functools

import jax
import jax.numpy as jnp
from jax import lax
from jax.experimental import pallas as pl
from jax.experimental.pallas import tpu as pltpu
from jax.experimental.pallas import tpu_sc as plsc

N_USERS = 1000000
N_ASPECTS = 8
BATCH = 16384
A_MIN, A_MAX = 1.0, 5.0
R_MIN, R_MAX = 1.0, 5.0

_NC = 2
_NS = 16
_NW = _NC * _NS
_BPW = BATCH // _NW
_ELEMS = _BPW * N_ASPECTS
_CHUNK = 128
_NCHUNK = _ELEMS // _CHUNK
_CPA = _BPW // _CHUNK
_GROUPS = _BPW // 16


@functools.partial(
    pl.kernel,
    mesh=plsc.VectorSubcoreMesh(core_axis_name="c", subcore_axis_name="s"),
    out_type=jax.ShapeDtypeStruct((BATCH,), jnp.float32),
    compiler_params=pltpu.CompilerParams(
        needs_layout_passes=False, use_tc_tiling_on_sc=False
    ),
    scratch_types=[
        pltpu.VMEM((_BPW,), jnp.int32),
        pltpu.VMEM((_ELEMS,), jnp.int32),
        pltpu.VMEM((_ELEMS,), jnp.float32),
        pltpu.VMEM((_ELEMS,), jnp.float32),
        pltpu.VMEM((_BPW,), jnp.float32),
        pltpu.SemaphoreType.DMA,
    ],
)
def _sc_predict(u_hbm, a_hbm, table_hbm, out_hbm, u_v, idx_v, w_v, a_v, out_v, sem):
    wid = lax.axis_index("s") * _NC + lax.axis_index("c")
    base = wid * _BPW

    pltpu.sync_copy(u_hbm.at[pl.ds(base, _BPW)], u_v)

    copies = []
    for c in range(_NCHUNK):
        j = c // _CPA
        ibase = (c % _CPA) * _CHUNK
        for t in range(_CHUNK // 16):
            u16 = u_v[pl.ds(ibase + t * 16, 16)]
            phys = (
                lax.shift_left(lax.shift_right_logical(u16, 7), 10)
                + (u16 & 127)
                + (j * 128)
            )
            idx_v[pl.ds(c * _CHUNK + t * 16, 16)] = phys
        copies.append(
            pltpu.async_copy(
                table_hbm.at[idx_v.at[pl.ds(c * _CHUNK, _CHUNK)]],
                w_v.at[pl.ds(c * _CHUNK, _CHUNK)],
                sem,
            )
        )
    for j in range(N_ASPECTS):
        pltpu.sync_copy(
            a_hbm.at[pl.ds(j * BATCH + base, _BPW)], a_v.at[pl.ds(j * _BPW, _BPW)]
        )
    for cp in copies:
        cp.wait()

    def group_body(g, _):
        o = g * 16
        acc = jnp.zeros((16,), jnp.float32)
        l1 = jnp.zeros((16,), jnp.float32)
        for j in range(N_ASPECTS):
            w = w_v[pl.ds(j * _BPW + o, 16)]
            a = a_v[pl.ds(j * _BPW + o, 16)]
            acc = acc + w * a
            l1 = l1 + jnp.abs(w)
        pred = acc / jnp.maximum(l1, 1e-12)
        out_v[pl.ds(o, 16)] = R_MIN + (R_MIN - R_MAX) * (
            (pred - A_MIN) / (A_MAX - A_MIN)
        )
        return 0

    lax.fori_loop(0, _GROUPS, group_body, 0)
    pltpu.sync_copy(out_v, out_hbm.at[pl.ds(base, _BPW)])


def kernel(U_ids, A_ratings, users_parameters):
    t = jnp.pad(users_parameters.T, ((0, 0), (0, 63)))
    t = t.reshape(N_ASPECTS, 7813, 128).transpose(1, 0, 2).reshape(-1)
    return _sc_predict(
        U_ids.astype(jnp.int32),
        A_ratings.reshape(-1),
        t,
    )


# speedup vs baseline: 10.2130x; 1.0807x over previous
import functools

import jax
import jax.numpy as jnp
from jax import lax
from jax.experimental import pallas as pl
from jax.experimental.pallas import tpu as pltpu
from jax.experimental.pallas import tpu_sc as plsc

N_USERS = 1000000
N_ASPECTS = 8
BATCH = 16384
A_MIN, A_MAX = 1.0, 5.0
R_MIN, R_MAX = 1.0, 5.0

_NC = 2
_NS = 16
_NW = _NC * _NS
_BPW = BATCH // _NW
_ELEMS = _BPW * N_ASPECTS
_CHUNK = 128
_NCHUNK = _ELEMS // _CHUNK
_CPA = _BPW // _CHUNK
_GROUPS = _BPW // 16
_TCHUNKS = (N_USERS + 64 + 127) // 128


@functools.partial(
    pl.kernel,
    mesh=plsc.VectorSubcoreMesh(core_axis_name="c", subcore_axis_name="s"),
    out_type=jax.ShapeDtypeStruct((BATCH,), jnp.float32),
    compiler_params=pltpu.CompilerParams(
        needs_layout_passes=False, use_tc_tiling_on_sc=False
    ),
    scratch_types=[
        pltpu.VMEM((_BPW,), jnp.int32),
        pltpu.VMEM((_ELEMS,), jnp.int32),
        pltpu.VMEM((_ELEMS,), jnp.float32),
        pltpu.VMEM((_ELEMS,), jnp.float32),
        pltpu.VMEM((_BPW,), jnp.float32),
        pltpu.SemaphoreType.DMA,
    ],
)
def _sc_predict(u_hbm, a_hbm, table_hbm, out_hbm, u_v, idx_v, w_v, a_v, out_v, sem):
    wid = lax.axis_index("s") * _NC + lax.axis_index("c")
    base = wid * _BPW

    pltpu.sync_copy(u_hbm.at[pl.ds(base, _BPW)], u_v)

    copies = []
    for c in range(_CPA):
        copies.append(
            pltpu.async_copy(
                a_hbm.at[pl.ds((_CPA * wid + c) * 1024, 1024)],
                a_v.at[pl.ds(c * 1024, 1024)],
                sem,
            )
        )

    pbase = []
    for t in range(_BPW // 16):
        u16 = u_v[pl.ds(t * 16, 16)]
        pbase.append(
            lax.shift_left(lax.shift_right_logical(u16, 7), 10) + (u16 & 127)
        )

    for c in range(_NCHUNK):
        j = c // _CPA
        tb = (c % _CPA) * (_CHUNK // 16)
        for t in range(_CHUNK // 16):
            idx_v[pl.ds(c * _CHUNK + t * 16, 16)] = pbase[tb + t] + (j * 128)
        copies.append(
            pltpu.async_copy(
                table_hbm.at[idx_v.at[pl.ds(c * _CHUNK, _CHUNK)]],
                w_v.at[pl.ds(c * _CHUNK, _CHUNK)],
                sem,
            )
        )
    for cp in copies:
        cp.wait()

    def group_body(g, _):
        o = g * 16
        ab = lax.div(g, 8) * 1024 + lax.rem(g, 8) * 16
        acc = jnp.zeros((16,), jnp.float32)
        l1 = jnp.zeros((16,), jnp.float32)
        for j in range(N_ASPECTS):
            w = w_v[pl.ds(j * _BPW + o, 16)]
            a = a_v[pl.ds(ab + j * 128, 16)]
            acc = acc + w * a
            l1 = l1 + jnp.abs(w)
        pred = acc / jnp.maximum(l1, 1e-12)
        out_v[pl.ds(o, 16)] = R_MIN + (R_MIN - R_MAX) * (
            (pred - A_MIN) / (A_MAX - A_MIN)
        )
        return 0

    lax.fori_loop(0, _GROUPS, group_body, 0)
    pltpu.sync_copy(out_v, out_hbm.at[pl.ds(base, _BPW)])


def kernel(U_ids, A_ratings, users_parameters):
    t = jnp.pad(users_parameters.T, ((0, 0), (0, 63)))
    t = t.reshape(N_ASPECTS, _TCHUNKS, 128).transpose(1, 0, 2).reshape(-1)
    a = A_ratings.reshape(N_ASPECTS, BATCH // 128, 128).transpose(1, 0, 2)
    return _sc_predict(U_ids.astype(jnp.int32), a.reshape(-1), t)
